# baseline (speedup 1.0000x reference)
import functools

import jax
import jax.numpy as jnp
from jax import lax
from jax.experimental import pallas as pl
from jax.experimental.pallas import tpu as pltpu
from jax.experimental.pallas import tpu_sc as plsc

EMB = 64
LANES = 16


def _mf_kernel(batch, num_workers):
    b_per_w = batch // num_workers
    n_groups = b_per_w // LANES
    mesh = plsc.VectorSubcoreMesh(core_axis_name="c", subcore_axis_name="s")
    num_cores = mesh.num_cores

    @functools.partial(
        pl.kernel,
        out_type=(
            jax.ShapeDtypeStruct((batch,), jnp.float32),
            jax.ShapeDtypeStruct((batch, EMB), jnp.float32),
            jax.ShapeDtypeStruct((batch, EMB), jnp.float32),
        ),
        mesh=mesh,
        compiler_params=pltpu.CompilerParams(
            needs_layout_passes=False, use_tc_tiling_on_sc=False),
        scratch_types=[
            pltpu.VMEM((b_per_w,), jnp.int32),
            pltpu.VMEM((b_per_w,), jnp.int32),
            pltpu.VMEM((b_per_w, EMB), jnp.float32),
            pltpu.VMEM((b_per_w, EMB), jnp.float32),
            pltpu.VMEM((b_per_w,), jnp.float32),
            pltpu.VMEM((b_per_w,), jnp.float32),
            pltpu.VMEM((LANES,), jnp.float32),
            pltpu.VMEM((b_per_w,), jnp.float32),
            pltpu.SemaphoreType.DMA,
            pltpu.SemaphoreType.DMA,
            pltpu.SemaphoreType.DMA,
            pltpu.SemaphoreType.DMA,
        ],
    )
    def k(u_id, i_id, user_emb, user_bias, item_emb, item_bias, mean,
          pred_out, u_out, i_out,
          uid_v, iid_v, u_rows, i_rows, ub_v, ib_v, mean_v, pred_v,
          sem_u, sem_i, sem_ub, sem_ib):
        wid = lax.axis_index("s") * num_cores + lax.axis_index("c")
        base = wid * b_per_w

        pltpu.sync_copy(u_id.at[pl.ds(base, b_per_w)], uid_v)
        pltpu.sync_copy(i_id.at[pl.ds(base, b_per_w)], iid_v)
        cp_u = pltpu.async_copy(user_emb.at[uid_v], u_rows, sem_u)
        cp_i = pltpu.async_copy(item_emb.at[iid_v], i_rows, sem_i)
        cp_ub = pltpu.async_copy(user_bias.at[uid_v], ub_v, sem_ub)
        cp_ib = pltpu.async_copy(item_bias.at[iid_v], ib_v, sem_ib)
        pltpu.sync_copy(mean, mean_v)

        cp_u.wait()
        pltpu.sync_copy(u_rows, u_out.at[pl.ds(base, b_per_w)])
        cp_i.wait()
        pltpu.sync_copy(i_rows, i_out.at[pl.ds(base, b_per_w)])
        cp_ub.wait()
        cp_ib.wait()

        mean_vec = mean_v[...]
        lane = jnp.arange(LANES, dtype=jnp.int32)

        def group(g, _):
            off = pl.multiple_of(g * LANES, LANES)
            r = jnp.zeros((LANES,), jnp.float32)
            for b in range(LANES):
                row = g * LANES + b
                acc = (u_rows[row, pl.ds(0, LANES)]
                       * i_rows[row, pl.ds(0, LANES)])
                for j in range(1, EMB // LANES):
                    acc = acc + (u_rows[row, pl.ds(j * LANES, LANES)]
                                 * i_rows[row, pl.ds(j * LANES, LANES)])
                s = jnp.sum(acc, axis=0)
                r = jnp.where(lane == b, s, r)
            pred_v[pl.ds(off, LANES)] = (
                r + ub_v[pl.ds(off, LANES)] + ib_v[pl.ds(off, LANES)]
                + mean_vec)
            return 0

        lax.fori_loop(0, n_groups, group, 0)
        pltpu.sync_copy(pred_v, pred_out.at[pl.ds(base, b_per_w)])

    return k


def kernel(u_id, i_id, user_emb, user_bias, item_emb, item_bias, mean):
    batch = u_id.shape[0]
    info = plsc.get_sparse_core_info()
    num_workers = info.num_cores * info.num_subcores
    k = _mf_kernel(batch, num_workers)
    pred, U, I = k(
        u_id.astype(jnp.int32),
        i_id.astype(jnp.int32),
        user_emb,
        jnp.reshape(user_bias, (-1,)),
        item_emb,
        jnp.reshape(item_bias, (-1,)),
        jnp.broadcast_to(mean, (LANES,)),
    )
    return (pred, U, I)
